# Initial kernel scaffold; baseline (speedup 1.0000x reference)
import jax
import jax.numpy as jnp
from jax.experimental import pallas as pl


def kernel(h_states, seq_start_end, end_pos, vx, vy, W_sp, b_sp, W_vel, b_vel, Wa1, ba1, ga1, bta1, Wa2, ba2, ga2, bta2, Wp1, bp1, gp1, btp1, Wp2, bp2, gp2, btp2):
    raise NotImplementedError("write your pallas kernel here")


# speedup vs baseline: 2.2011x; 2.2011x over previous
import functools

import jax
import jax.numpy as jnp
from jax.experimental import pallas as pl

S, P, H, EMB = 128, 16, 64, 64
D1, D2 = 512, 1024
NS = 4


def _pool_kernel(pos_ref, h_ref, A_ref, Df_ref, c_ref, W2_ref, b2_ref, out_ref):
    pos = pos_ref[...]
    h = h_ref[...]
    uv = jnp.dot(pos, A_ref[...], preferred_element_type=jnp.float32)
    u = 0.05 * (uv + jnp.dot(h, Df_ref[...], preferred_element_type=jnp.float32))
    u = u + c_ref[...]
    v = 0.05 * uv
    u4 = u.reshape(NS, 1, P, D1)
    v4 = v.reshape(NS, P, 1, D1)
    x1 = jnp.maximum(u4 - v4, 0.0)
    x1 = x1.reshape(NS * P * P, D1).astype(jnp.bfloat16)
    z = jnp.dot(x1, W2_ref[...], preferred_element_type=jnp.float32)
    z = z.reshape(NS * P, P, D2).max(axis=1)
    out_ref[...] = jnp.maximum(z + b2_ref[...], 0.0)


@functools.partial(jax.jit, static_argnames=())
def kernel(h_states, seq_start_end, end_pos, vx, vy,
           W_sp, b_sp, W_vel, b_vel,
           Wa1, ba1, ga1, bta1, Wa2, ba2, ga2, bta2,
           Wp1, bp1, gp1, btp1, Wp2, bp2, gp2, btp2):
    B = end_pos.shape[0]
    inv = 1.0 / jnp.sqrt(1.0 + 1e-5)
    s1 = gp1 * inv
    W1f = Wp1 * s1[None, :]
    b1f = bp1 * s1 + btp1
    A = W_sp @ W1f[:EMB]
    Df = W1f[EMB:]
    c = (0.05 * (b_sp @ W1f[:EMB]) + b1f).reshape(1, D1)
    s2 = gp2 * inv
    W2f = (Wp2 * s2[None, :]).astype(jnp.bfloat16)
    b2f = (bp2 * s2 + btp2).reshape(1, D2)

    h = h_states.reshape(B, H)
    blk = NS * P
    grid = (S // NS,)
    out = pl.pallas_call(
        _pool_kernel,
        grid=grid,
        in_specs=[
            pl.BlockSpec((blk, 2), lambda i: (i, 0)),
            pl.BlockSpec((blk, H), lambda i: (i, 0)),
            pl.BlockSpec((2, D1), lambda i: (0, 0)),
            pl.BlockSpec((H, D1), lambda i: (0, 0)),
            pl.BlockSpec((1, D1), lambda i: (0, 0)),
            pl.BlockSpec((D1, D2), lambda i: (0, 0)),
            pl.BlockSpec((1, D2), lambda i: (0, 0)),
        ],
        out_specs=pl.BlockSpec((blk, D2), lambda i: (i, 0)),
        out_shape=jax.ShapeDtypeStruct((B, D2), jnp.float32),
    )(end_pos, h, A, Df, c, W2f, b2f)
    return out


# speedup vs baseline: 2.6974x; 1.2255x over previous
import functools

import jax
import jax.numpy as jnp
from jax.experimental import pallas as pl

S, P, H, EMB = 128, 16, 64, 64
D1, D2 = 512, 1024
NS = 16


def _pool_kernel(pos_ref, h_ref, A_ref, Df_ref, c_ref, W2_ref, b2_ref, out_ref):
    pos = pos_ref[...]
    h = h_ref[...]
    uv = jnp.dot(pos, A_ref[...], preferred_element_type=jnp.float32)
    u = 0.05 * (uv + jnp.dot(h, Df_ref[...], preferred_element_type=jnp.float32))
    u = (u + c_ref[...]).astype(jnp.bfloat16)
    v = (0.05 * uv).astype(jnp.bfloat16)
    u3 = u.reshape(NS, P, D1)
    v3 = v.reshape(NS, P, D1)
    W2 = W2_ref[...]
    acc = None
    for j in range(P):
        x1 = jnp.maximum(u3[:, j:j + 1, :] - v3, 0).reshape(NS * P, D1)
        zj = jnp.dot(x1, W2, preferred_element_type=jnp.float32)
        acc = zj if acc is None else jnp.maximum(acc, zj)
    out_ref[...] = jnp.maximum(acc + b2_ref[...], 0.0)


@functools.partial(jax.jit, static_argnames=())
def kernel(h_states, seq_start_end, end_pos, vx, vy,
           W_sp, b_sp, W_vel, b_vel,
           Wa1, ba1, ga1, bta1, Wa2, ba2, ga2, bta2,
           Wp1, bp1, gp1, btp1, Wp2, bp2, gp2, btp2):
    B = end_pos.shape[0]
    inv = 1.0 / jnp.sqrt(1.0 + 1e-5)
    s1 = gp1 * inv
    W1f = Wp1 * s1[None, :]
    b1f = bp1 * s1 + btp1
    A = W_sp @ W1f[:EMB]
    Df = W1f[EMB:]
    c = (0.05 * (b_sp @ W1f[:EMB]) + b1f).reshape(1, D1)
    s2 = gp2 * inv
    W2f = (Wp2 * s2[None, :]).astype(jnp.bfloat16)
    b2f = (bp2 * s2 + btp2).reshape(1, D2)

    h = h_states.reshape(B, H)
    blk = NS * P
    grid = (S // NS,)
    out = pl.pallas_call(
        _pool_kernel,
        grid=grid,
        in_specs=[
            pl.BlockSpec((blk, 2), lambda i: (i, 0)),
            pl.BlockSpec((blk, H), lambda i: (i, 0)),
            pl.BlockSpec((2, D1), lambda i: (0, 0)),
            pl.BlockSpec((H, D1), lambda i: (0, 0)),
            pl.BlockSpec((1, D1), lambda i: (0, 0)),
            pl.BlockSpec((D1, D2), lambda i: (0, 0)),
            pl.BlockSpec((1, D2), lambda i: (0, 0)),
        ],
        out_specs=pl.BlockSpec((blk, D2), lambda i: (i, 0)),
        out_shape=jax.ShapeDtypeStruct((B, D2), jnp.float32),
    )(end_pos, h, A, Df, c, W2f, b2f)
    return out


# speedup vs baseline: 2.7270x; 1.0110x over previous
import functools

import jax
import jax.numpy as jnp
from jax.experimental import pallas as pl

S, P, H, EMB = 128, 16, 64, 64
D1, D2 = 512, 1024
NS = 16


def _pool_kernel(pos_ref, h_ref, A_ref, Df_ref, c_ref, W2_ref, b2_ref, out_ref):
    pos = pos_ref[...]
    h = h_ref[...]
    uv = jnp.dot(pos, A_ref[...], preferred_element_type=jnp.float32)
    u = 0.05 * (uv + jnp.dot(h, Df_ref[...], preferred_element_type=jnp.float32))
    u = (u + c_ref[...]).astype(jnp.bfloat16)
    v = (0.05 * uv).astype(jnp.bfloat16)
    u3 = u.reshape(NS, P, D1)
    v3 = v.reshape(NS, P, D1)
    x1 = jnp.concatenate(
        [jnp.maximum(u3[:, j:j + 1, :] - v3, 0).reshape(NS * P, D1)
         for j in range(P)], axis=0)
    z = jnp.dot(x1, W2_ref[...], preferred_element_type=jnp.float32)
    acc = z[:NS * P]
    for j in range(1, P):
        acc = jnp.maximum(acc, z[j * NS * P:(j + 1) * NS * P])
    out_ref[...] = jnp.maximum(acc + b2_ref[...], 0.0)


@functools.partial(jax.jit, static_argnames=())
def kernel(h_states, seq_start_end, end_pos, vx, vy,
           W_sp, b_sp, W_vel, b_vel,
           Wa1, ba1, ga1, bta1, Wa2, ba2, ga2, bta2,
           Wp1, bp1, gp1, btp1, Wp2, bp2, gp2, btp2):
    B = end_pos.shape[0]
    inv = 1.0 / jnp.sqrt(1.0 + 1e-5)
    s1 = gp1 * inv
    W1f = Wp1 * s1[None, :]
    b1f = bp1 * s1 + btp1
    A = W_sp @ W1f[:EMB]
    Df = W1f[EMB:]
    c = (0.05 * (b_sp @ W1f[:EMB]) + b1f).reshape(1, D1)
    s2 = gp2 * inv
    W2f = (Wp2 * s2[None, :]).astype(jnp.bfloat16)
    b2f = (bp2 * s2 + btp2).reshape(1, D2)

    h = h_states.reshape(B, H)
    blk = NS * P
    grid = (S // NS,)
    out = pl.pallas_call(
        _pool_kernel,
        grid=grid,
        in_specs=[
            pl.BlockSpec((blk, 2), lambda i: (i, 0)),
            pl.BlockSpec((blk, H), lambda i: (i, 0)),
            pl.BlockSpec((2, D1), lambda i: (0, 0)),
            pl.BlockSpec((H, D1), lambda i: (0, 0)),
            pl.BlockSpec((1, D1), lambda i: (0, 0)),
            pl.BlockSpec((D1, D2), lambda i: (0, 0)),
            pl.BlockSpec((1, D2), lambda i: (0, 0)),
        ],
        out_specs=pl.BlockSpec((blk, D2), lambda i: (i, 0)),
        out_shape=jax.ShapeDtypeStruct((B, D2), jnp.float32),
    )(end_pos, h, A, Df, c, W2f, b2f)
    return out


# speedup vs baseline: 3.1062x; 1.1391x over previous
import jax
import jax.numpy as jnp
from jax.experimental import pallas as pl
from jax.experimental.pallas import tpu as pltpu

S, P, H, EMB = 128, 16, 64, 64
D1, D2 = 512, 1024
NS = 16


def _pool_kernel(pos_ref, h_ref, Wsp_ref, bsp_ref,
                 Wp1_ref, bp1_ref, gp1_ref, btp1_ref,
                 Wp2_ref, bp2_ref, gp2_ref, btp2_ref,
                 out_ref, A_s, Df_s, c_s, W2_s, b2_s):
    @pl.when(pl.program_id(0) == 0)
    def _prep():
        inv = 1.0 / jnp.sqrt(1.0 + 1e-5)
        s1 = gp1_ref[...] * inv
        W1t = Wp1_ref[:EMB] * s1
        A_s[...] = jnp.dot(Wsp_ref[...], W1t, preferred_element_type=jnp.float32)
        Df_s[...] = Wp1_ref[EMB:] * s1
        c_s[...] = (0.05 * jnp.dot(bsp_ref[...], W1t,
                                   preferred_element_type=jnp.float32)
                    + bp1_ref[...] * s1 + btp1_ref[...])
        s2 = gp2_ref[...] * inv
        W2_s[...] = (Wp2_ref[...] * s2).astype(jnp.bfloat16)
        b2_s[...] = bp2_ref[...] * s2 + btp2_ref[...]

    pos = pos_ref[...]
    h = h_ref[...]
    uv = jnp.dot(pos, A_s[...], preferred_element_type=jnp.float32)
    u = 0.05 * (uv + jnp.dot(h, Df_s[...], preferred_element_type=jnp.float32))
    u = (u + c_s[...]).astype(jnp.bfloat16)
    v = (0.05 * uv).astype(jnp.bfloat16)
    u3 = u.reshape(NS, P, D1)
    v3 = v.reshape(NS, P, D1)
    x1 = jnp.concatenate(
        [jnp.maximum(u3[:, j:j + 1, :] - v3, 0).reshape(NS * P, D1)
         for j in range(P)], axis=0)
    z = jnp.dot(x1, W2_s[...], preferred_element_type=jnp.float32)
    acc = z[:NS * P]
    for j in range(1, P):
        acc = jnp.maximum(acc, z[j * NS * P:(j + 1) * NS * P])
    out_ref[...] = jnp.maximum(acc + b2_s[...], 0.0)


@jax.jit
def kernel(h_states, seq_start_end, end_pos, vx, vy,
           W_sp, b_sp, W_vel, b_vel,
           Wa1, ba1, ga1, bta1, Wa2, ba2, ga2, bta2,
           Wp1, bp1, gp1, btp1, Wp2, bp2, gp2, btp2):
    B = end_pos.shape[0]
    h = h_states.reshape(B, H)
    blk = NS * P
    row = lambda a: a.reshape(1, -1)
    whole = lambda shp: pl.BlockSpec(shp, lambda i: (0, 0))
    out = pl.pallas_call(
        _pool_kernel,
        grid=(S // NS,),
        in_specs=[
            pl.BlockSpec((blk, 2), lambda i: (i, 0)),
            pl.BlockSpec((blk, H), lambda i: (i, 0)),
            whole((2, EMB)), whole((1, EMB)),
            whole((2 * EMB, D1)), whole((1, D1)), whole((1, D1)), whole((1, D1)),
            whole((D1, D2)), whole((1, D2)), whole((1, D2)), whole((1, D2)),
        ],
        out_specs=pl.BlockSpec((blk, D2), lambda i: (i, 0)),
        out_shape=jax.ShapeDtypeStruct((B, D2), jnp.float32),
        scratch_shapes=[
            pltpu.VMEM((2, D1), jnp.float32),
            pltpu.VMEM((EMB, D1), jnp.float32),
            pltpu.VMEM((1, D1), jnp.float32),
            pltpu.VMEM((D1, D2), jnp.bfloat16),
            pltpu.VMEM((1, D2), jnp.float32),
        ],
    )(end_pos, h, W_sp, row(b_sp), Wp1, row(bp1), row(gp1), row(btp1),
      Wp2, row(bp2), row(gp2), row(btp2))
    return out
